# baseline (speedup 1.0000x reference)
import jax
import jax.numpy as jnp
from jax import lax
from jax.experimental import pallas as pl
from jax.experimental.pallas import tpu as pltpu
from jax.experimental.pallas import tpu_sc as plsc

N = 10000
E = 160000
L = 20
V = 50000
D = 256
MQ = 32
NG = D // MQ

NP = 10240
NB = NP // 16
C = 128
TRASH = NP - 1

ECH = 80
EPT = ECH * C
EP = EPT * 16

WCH = 104
WPT = WCH * C

G4 = 4

_mesh = plsc.VectorSubcoreMesh(core_axis_name="c", subcore_axis_name="s")


def _zero_shared(zhbm, stage, shared, base, nrows):
    pltpu.sync_copy(zhbm, stage)
    for j in range(nrows // C):
        pltpu.sync_copy(stage, shared.at[pl.ds(base + j * C, C)])


def _copy_out_shared(shared, base, nrows, stage, out):
    for j in range(nrows // C):
        pltpu.sync_copy(shared.at[pl.ds(base + j * C, C)], stage)
        pltpu.sync_copy(stage, out.at[pl.ds(base + j * C, C)])


def _segsum_round(c, src_a, src_b, zm, sidx_all, didx_all, rows, acc,
                  semg, sems, nch):
    def group(g, p):
        @pl.when(g >= 2)
        def _():
            for j in range(G4):
                pltpu.make_async_copy(zm, rows[p * G4 + j], sems).wait()

        for j in range(G4):
            k = g * G4 + j

            @pl.when(c == 0)
            def _():
                pltpu.async_copy(src_a.at[sidx_all.at[k]],
                                 rows[p * G4 + j], semg)

            @pl.when(c == 1)
            def _():
                pltpu.async_copy(src_b.at[sidx_all.at[k]],
                                 rows[p * G4 + j], semg)

        for j in range(G4):
            pltpu.make_async_copy(zm, rows[p * G4 + j], semg).wait()
        for j in range(G4):
            k = g * G4 + j
            pltpu.async_copy(rows[p * G4 + j], acc.at[didx_all.at[k]],
                             sems, add=True)

    def body(i, carry):
        group(2 * i, 0)
        group(2 * i + 1, 1)
        return carry

    lax.fori_loop(0, nch // (2 * G4), body, 0)
    for j in range(2 * G4):
        pltpu.make_async_copy(zm, rows[j], sems).wait()


def _embdeg_body(t0, t1, t2, t3, t4, t5, t6, t7, wids3, nids3,
                 esrc3, edst3, zm, ones32,
                 we0, we1, we2, we3, we4, we5, we6, we7, dsrc, ddst,
                 widx_all, nidx_all, eidx_all,
                 r0, r1, r2, r3, r4, r5, r6, r7,
                 obuf, acc, semg, sems):
    c = lax.axis_index("c")
    s = lax.axis_index("s")
    base_n = s * NB
    rows = (r0, r1, r2, r3, r4, r5, r6, r7)
    pltpu.sync_copy(wids3.at[s], widx_all)
    pltpu.sync_copy(nids3.at[s], nidx_all)

    for ta, tb, oa, ob in ((t0, t4, we0, we4), (t1, t5, we1, we5),
                           (t2, t6, we2, we6), (t3, t7, we3, we7)):
        _zero_shared(zm, rows[0], acc, base_n, NB)
        plsc.subcore_barrier()
        _segsum_round(c, ta, tb, zm, widx_all, nidx_all, rows, acc,
                      semg, sems, WCH)
        plsc.subcore_barrier()

        @pl.when(c == 0)
        def _():
            _copy_out_shared(acc, base_n, NB, rows[0], oa)

        @pl.when(c == 1)
        def _():
            _copy_out_shared(acc, base_n, NB, rows[0], ob)

    @pl.when(c == 0)
    def _():
        pltpu.sync_copy(esrc3.at[s], eidx_all)

    @pl.when(c == 1)
    def _():
        pltpu.sync_copy(edst3.at[s], eidx_all)

    pltpu.sync_copy(ones32, obuf)
    _zero_shared(zm, rows[0], acc, base_n, NB)
    plsc.subcore_barrier()

    def deg_step(i, carry):
        for j in range(G4):
            k = i * G4 + j
            pltpu.async_copy(obuf, acc.at[eidx_all.at[k]], sems, add=True)

        @pl.when(i >= 1)
        def _():
            for j in range(G4):
                pltpu.make_async_copy(zm, obuf, sems).wait()

        return carry

    lax.fori_loop(0, ECH // G4, deg_step, 0)
    for j in range(G4):
        pltpu.make_async_copy(zm, obuf, sems).wait()

    plsc.subcore_barrier()

    @pl.when(c == 0)
    def _():
        _copy_out_shared(acc, base_n, NB, rows[0], dsrc)

    @pl.when(c == 1)
    def _():
        _copy_out_shared(acc, base_n, NB, rows[0], ddst)


_embdeg = pl.kernel(
    _embdeg_body,
    out_type=tuple(jax.ShapeDtypeStruct((NP, MQ), jnp.float32)
                   for _ in range(10)),
    mesh=_mesh,
    compiler_params=pltpu.CompilerParams(use_tc_tiling_on_sc=False),
    scratch_types=[
        pltpu.VMEM((WCH, C), jnp.int32),
        pltpu.VMEM((WCH, C), jnp.int32),
        pltpu.VMEM((ECH, C), jnp.int32),
    ] + [pltpu.VMEM((C, MQ), jnp.float32) for _ in range(9)] + [
        pltpu.VMEM_SHARED((NP, MQ), jnp.float32),
        pltpu.SemaphoreType.DMA,
        pltpu.SemaphoreType.DMA,
    ],
)


def _msg_body(x0, x1, x2, x3, x4, x5, x6, x7, esrc3, edst3, zm,
              m0, m1, m2, m3, m4, m5, m6, m7,
              sidx_all, didx_all,
              r0, r1, r2, r3, r4, r5, r6, r7,
              acc, semg, sems):
    c = lax.axis_index("c")
    s = lax.axis_index("s")
    base_n = s * NB
    rows = (r0, r1, r2, r3, r4, r5, r6, r7)
    pltpu.sync_copy(esrc3.at[s], sidx_all)
    pltpu.sync_copy(edst3.at[s], didx_all)

    for xa, xb, oa, ob in ((x0, x4, m0, m4), (x1, x5, m1, m5),
                           (x2, x6, m2, m6), (x3, x7, m3, m7)):
        _zero_shared(zm, rows[0], acc, base_n, NB)
        plsc.subcore_barrier()
        _segsum_round(c, xa, xb, zm, sidx_all, didx_all, rows, acc,
                      semg, sems, ECH)
        plsc.subcore_barrier()

        @pl.when(c == 0)
        def _():
            _copy_out_shared(acc, base_n, NB, rows[0], oa)

        @pl.when(c == 1)
        def _():
            _copy_out_shared(acc, base_n, NB, rows[0], ob)


_msg = pl.kernel(
    _msg_body,
    out_type=tuple(jax.ShapeDtypeStruct((NP, MQ), jnp.float32)
                   for _ in range(8)),
    mesh=_mesh,
    compiler_params=pltpu.CompilerParams(use_tc_tiling_on_sc=False),
    scratch_types=[
        pltpu.VMEM((ECH, C), jnp.int32),
        pltpu.VMEM((ECH, C), jnp.int32),
    ] + [pltpu.VMEM((C, MQ), jnp.float32) for _ in range(8)] + [
        pltpu.VMEM_SHARED((NP, MQ), jnp.float32),
        pltpu.SemaphoreType.DMA,
        pltpu.SemaphoreType.DMA,
    ],
)

B = 640
GRID = NP // B

_eighth_spec = pl.BlockSpec((B, MQ), lambda i: (i, 0))


def _tc1_body(we0_ref, we1_ref, we2_ref, we3_ref, we4_ref, we5_ref,
              we6_ref, we7_ref, scal_ref, dsrc_ref, w_ref, *o_refs):
    h = jnp.concatenate([we0_ref[...], we1_ref[...], we2_ref[...],
                         we3_ref[...], we4_ref[...], we5_ref[...],
                         we6_ref[...], we7_ref[...]], axis=1)
    ml = scal_ref[:, 0:1]
    f = scal_ref[:, 1:2]
    lf = scal_ref[:, 2:3]
    ll = scal_ref[:, 3:4]
    col = lax.broadcasted_iota(jnp.int32, (B, D), 1)
    h = h / ml
    h = jnp.where(col == D - 3, f, h)
    h = jnp.where(col == D - 2, lf, h)
    h = jnp.where(col == D - 1, ll, h)
    sout = lax.rsqrt(jnp.maximum(dsrc_ref[:, 0:1], 1.0))
    x = jnp.dot(h * sout, w_ref[...], preferred_element_type=jnp.float32)
    for j, o_ref in enumerate(o_refs):
        o_ref[...] = x[:, j * MQ:(j + 1) * MQ]


_tc1 = pl.pallas_call(
    _tc1_body,
    grid=(GRID,),
    in_specs=[_eighth_spec] * 8 + [
        pl.BlockSpec((B, 4), lambda i: (i, 0)),
        _eighth_spec,
        pl.BlockSpec((D, D), lambda i: (0, 0)),
    ],
    out_specs=[_eighth_spec] * 8,
    out_shape=[jax.ShapeDtypeStruct((NP, MQ), jnp.float32)] * 8,
)


def _tc2_body(m0_ref, m1_ref, m2_ref, m3_ref, m4_ref, m5_ref, m6_ref,
              m7_ref, dsrc_ref, ddst_ref, b_ref, w_ref, *o_refs):
    agg = jnp.concatenate([m0_ref[...], m1_ref[...], m2_ref[...],
                           m3_ref[...], m4_ref[...], m5_ref[...],
                           m6_ref[...], m7_ref[...]], axis=1)
    sin = lax.rsqrt(jnp.maximum(ddst_ref[:, 0:1], 1.0))
    sout = lax.rsqrt(jnp.maximum(dsrc_ref[:, 0:1], 1.0))
    h = jnp.maximum(agg * sin + b_ref[...], 0.0)
    x = jnp.dot(h * sout, w_ref[...], preferred_element_type=jnp.float32)
    for j, o_ref in enumerate(o_refs):
        o_ref[...] = x[:, j * MQ:(j + 1) * MQ]


_tc2 = pl.pallas_call(
    _tc2_body,
    grid=(GRID,),
    in_specs=[_eighth_spec] * 8 + [
        _eighth_spec,
        _eighth_spec,
        pl.BlockSpec((1, D), lambda i: (0, 0)),
        pl.BlockSpec((D, D), lambda i: (0, 0)),
    ],
    out_specs=[_eighth_spec] * 8,
    out_shape=[jax.ShapeDtypeStruct((NP, MQ), jnp.float32)] * 8,
)


def _tc3_body(m0_ref, m1_ref, m2_ref, m3_ref, m4_ref, m5_ref, m6_ref,
              m7_ref, ddst_ref, b_ref, oh_ref, og_ref):
    agg = jnp.concatenate([m0_ref[...], m1_ref[...], m2_ref[...],
                           m3_ref[...], m4_ref[...], m5_ref[...],
                           m6_ref[...], m7_ref[...]], axis=1)
    sin = lax.rsqrt(jnp.maximum(ddst_ref[:, 0:1], 1.0))
    h = jnp.maximum(agg * sin + b_ref[...], 0.0)
    oh_ref[...] = h
    i = pl.program_id(0)
    rows = lax.broadcasted_iota(jnp.int32, (B, 1), 0) + i * B
    part = jnp.sum(jnp.where(rows < N, h, 0.0), axis=0, keepdims=True)

    @pl.when(i == 0)
    def _():
        og_ref[...] = jnp.zeros_like(og_ref)

    og_ref[...] += part

    @pl.when(i == GRID - 1)
    def _():
        og_ref[...] = og_ref[...] * (1.0 / N)


_tc3 = pl.pallas_call(
    _tc3_body,
    grid=(GRID,),
    in_specs=[_eighth_spec] * 8 + [
        _eighth_spec,
        pl.BlockSpec((1, D), lambda i: (0, 0)),
    ],
    out_specs=[
        pl.BlockSpec((B, D), lambda i: (i, 0)),
        pl.BlockSpec((1, D), lambda i: (0, 0)),
    ],
    out_shape=[
        jax.ShapeDtypeStruct((NP, D), jnp.float32),
        jax.ShapeDtypeStruct((1, D), jnp.float32),
    ],
)


@jax.jit
def kernel(word_ids, ml, f, lf, ll, edge_index, emb_table, W1, b1, W2, b2):
    t = jnp.pad(emb_table, ((0, 0), (0, D - emb_table.shape[1])))
    tq = [t[:, i * MQ:(i + 1) * MQ] for i in range(NG)]
    wt = jnp.concatenate(
        [word_ids.astype(jnp.int32).reshape(-1),
         jnp.ones(((NP - N) * L,), jnp.int32)]).reshape(16, NB * L)
    wids3 = jnp.concatenate(
        [wt, jnp.ones((16, WPT - NB * L), jnp.int32)], axis=1)
    wids3 = wids3.reshape(16, WCH, C)
    nt = (jnp.arange(NP * L, dtype=jnp.int32) // L).reshape(16, NB * L)
    nids3 = jnp.concatenate(
        [nt, jnp.full((16, WPT - NB * L), TRASH, jnp.int32)], axis=1)
    nids3 = nids3.reshape(16, WCH, C)
    ei = edge_index.astype(jnp.int32)
    pad_e = jnp.full((EP - E,), TRASH, jnp.int32)
    esrc3 = jnp.concatenate([ei[0], pad_e]).reshape(16, ECH, C)
    edst3 = jnp.concatenate([ei[1], pad_e]).reshape(16, ECH, C)
    scal = jnp.stack([ml, f, lf, ll], axis=1)
    scal = jnp.concatenate(
        [scal,
         jnp.concatenate([jnp.ones((NP - N, 1), jnp.float32),
                          jnp.zeros((NP - N, 3), jnp.float32)], axis=1)])
    zm = jnp.zeros((C, MQ), jnp.float32)
    ones32 = jnp.ones((C, MQ), jnp.float32)

    o = _embdeg(*tq, wids3, nids3, esrc3, edst3, zm, ones32)
    we = o[:8]
    dsrc, ddst = o[8], o[9]
    x = _tc1(*we, scal, dsrc, W1)
    m = _msg(*x, esrc3, edst3, zm)
    y = _tc2(*m, dsrc, ddst, b1.reshape(1, D), W2)
    n = _msg(*y, esrc3, edst3, zm)
    h, hg = _tc3(*n, ddst, b2.reshape(1, D))
    return h[:N], hg


# speedup vs baseline: 1.0708x; 1.0708x over previous
import jax
import jax.numpy as jnp
from jax import lax
from jax.experimental import pallas as pl
from jax.experimental.pallas import tpu as pltpu
from jax.experimental.pallas import tpu_sc as plsc

N = 10000
E = 160000
L = 20
V = 50000
D = 256
MQ = 32
NG = D // MQ

NP = 10240
NB = NP // 16
C = 128
TRASH = NP - 1

ECH = 80
EPT = ECH * C
EP = EPT * 16

WCH = 104
WPT = WCH * C

G4 = 4

_mesh = plsc.VectorSubcoreMesh(core_axis_name="c", subcore_axis_name="s")


def _zero_shared(zhbm, stage, shared, base, nrows):
    pltpu.sync_copy(zhbm, stage)
    for j in range(nrows // C):
        pltpu.sync_copy(stage, shared.at[pl.ds(base + j * C, C)])


def _copy_out_shared(shared, base, nrows, stage, out):
    for j in range(nrows // C):
        pltpu.sync_copy(shared.at[pl.ds(base + j * C, C)], stage)
        pltpu.sync_copy(stage, out.at[pl.ds(base + j * C, C)])


def _segsum_round(c, src_a, src_b, zm, sidx_all, didx_all, rows, acc,
                  semg, sems, nch):
    def group(g, p):
        @pl.when(g >= 2)
        def _():
            for j in range(G4):
                pltpu.make_async_copy(zm, rows[p * G4 + j], sems).wait()

        for j in range(G4):
            k = g * G4 + j

            @pl.when(c == 0)
            def _():
                pltpu.async_copy(src_a.at[sidx_all.at[k]],
                                 rows[p * G4 + j], semg)

            @pl.when(c == 1)
            def _():
                pltpu.async_copy(src_b.at[sidx_all.at[k]],
                                 rows[p * G4 + j], semg)

        for j in range(G4):
            pltpu.make_async_copy(zm, rows[p * G4 + j], semg).wait()
        for j in range(G4):
            k = g * G4 + j
            pltpu.async_copy(rows[p * G4 + j], acc.at[didx_all.at[k]],
                             sems, add=True)

    def body(i, carry):
        group(2 * i, 0)
        group(2 * i + 1, 1)
        return carry

    lax.fori_loop(0, nch // (2 * G4), body, 0)
    for j in range(2 * G4):
        pltpu.make_async_copy(zm, rows[j], sems).wait()


def _embdeg_body(t0, t1, t2, t3, t4, t5, t6, t7, wids3, nids3,
                 esrc3, edst3, zm, ones32,
                 we0, we1, we2, we3, we4, we5, we6, we7, dsrc, ddst,
                 widx_all, nidx_all, eidx_all,
                 r0, r1, r2, r3, r4, r5, r6, r7,
                 obuf, acc, semg, sems):
    c = lax.axis_index("c")
    s = lax.axis_index("s")
    base_n = s * NB
    rows = (r0, r1, r2, r3, r4, r5, r6, r7)
    pltpu.sync_copy(wids3.at[s], widx_all)
    pltpu.sync_copy(nids3.at[s], nidx_all)

    for ta, tb, oa, ob in ((t0, t4, we0, we4), (t1, t5, we1, we5),
                           (t2, t6, we2, we6), (t3, t7, we3, we7)):
        _zero_shared(zm, rows[0], acc, base_n, NB)
        plsc.subcore_barrier()
        _segsum_round(c, ta, tb, zm, widx_all, nidx_all, rows, acc,
                      semg, sems, WCH)
        plsc.subcore_barrier()

        @pl.when(c == 0)
        def _():
            _copy_out_shared(acc, base_n, NB, rows[0], oa)

        @pl.when(c == 1)
        def _():
            _copy_out_shared(acc, base_n, NB, rows[0], ob)

    @pl.when(c == 0)
    def _():
        pltpu.sync_copy(esrc3.at[s], eidx_all)

    @pl.when(c == 1)
    def _():
        pltpu.sync_copy(edst3.at[s], eidx_all)

    pltpu.sync_copy(ones32, obuf)
    _zero_shared(zm, rows[0], acc, base_n, NB)
    plsc.subcore_barrier()

    def deg_step(i, carry):
        for j in range(G4):
            k = i * G4 + j
            pltpu.async_copy(obuf, acc.at[eidx_all.at[k]], sems, add=True)

        @pl.when(i >= 1)
        def _():
            for j in range(G4):
                pltpu.make_async_copy(zm, obuf, sems).wait()

        return carry

    lax.fori_loop(0, ECH // G4, deg_step, 0)
    for j in range(G4):
        pltpu.make_async_copy(zm, obuf, sems).wait()

    plsc.subcore_barrier()

    @pl.when(c == 0)
    def _():
        _copy_out_shared(acc, base_n, NB, rows[0], dsrc)

    @pl.when(c == 1)
    def _():
        _copy_out_shared(acc, base_n, NB, rows[0], ddst)


_embdeg = pl.kernel(
    _embdeg_body,
    out_type=tuple(jax.ShapeDtypeStruct((NP, MQ), jnp.float32)
                   for _ in range(10)),
    mesh=_mesh,
    compiler_params=pltpu.CompilerParams(use_tc_tiling_on_sc=False),
    scratch_types=[
        pltpu.VMEM((WCH, C), jnp.int32),
        pltpu.VMEM((WCH, C), jnp.int32),
        pltpu.VMEM((ECH, C), jnp.int32),
    ] + [pltpu.VMEM((C, MQ), jnp.float32) for _ in range(9)] + [
        pltpu.VMEM_SHARED((NP, MQ), jnp.float32),
        pltpu.SemaphoreType.DMA,
        pltpu.SemaphoreType.DMA,
    ],
)


def _msg_body(x0, x1, x2, x3, x4, x5, x6, x7, esrc3, edst3, zm,
              m0, m1, m2, m3, m4, m5, m6, m7,
              sidx_all, didx_all,
              r0, r1, r2, r3, r4, r5, r6, r7,
              acc, semg, sems):
    c = lax.axis_index("c")
    s = lax.axis_index("s")
    base_n = s * NB
    rows = (r0, r1, r2, r3, r4, r5, r6, r7)
    pltpu.sync_copy(esrc3.at[s], sidx_all)
    pltpu.sync_copy(edst3.at[s], didx_all)

    for xa, xb, oa, ob in ((x0, x4, m0, m4), (x1, x5, m1, m5),
                           (x2, x6, m2, m6), (x3, x7, m3, m7)):
        _zero_shared(zm, rows[0], acc, base_n, NB)
        plsc.subcore_barrier()
        _segsum_round(c, xa, xb, zm, sidx_all, didx_all, rows, acc,
                      semg, sems, ECH)
        plsc.subcore_barrier()

        @pl.when(c == 0)
        def _():
            _copy_out_shared(acc, base_n, NB, rows[0], oa)

        @pl.when(c == 1)
        def _():
            _copy_out_shared(acc, base_n, NB, rows[0], ob)


_msg = pl.kernel(
    _msg_body,
    out_type=tuple(jax.ShapeDtypeStruct((NP, MQ), jnp.float32)
                   for _ in range(8)),
    mesh=_mesh,
    compiler_params=pltpu.CompilerParams(use_tc_tiling_on_sc=False),
    scratch_types=[
        pltpu.VMEM((ECH, C), jnp.int32),
        pltpu.VMEM((ECH, C), jnp.int32),
    ] + [pltpu.VMEM((C, MQ), jnp.float32) for _ in range(8)] + [
        pltpu.VMEM_SHARED((NP, MQ), jnp.float32),
        pltpu.SemaphoreType.DMA,
        pltpu.SemaphoreType.DMA,
    ],
)


def _msg64_body(x0, x1, x2, x3, esrc3, edst3, zq,
                m0, m1, m2, m3,
                sidx_all, didx_all,
                r0, r1, r2, r3, r4, r5, r6, r7,
                acc, semg, sems):
    c = lax.axis_index("c")
    s = lax.axis_index("s")
    base_n = s * NB
    rows = (r0, r1, r2, r3, r4, r5, r6, r7)
    pltpu.sync_copy(esrc3.at[s], sidx_all)
    pltpu.sync_copy(edst3.at[s], didx_all)

    for xa, xb, oa, ob in ((x0, x2, m0, m2), (x1, x3, m1, m3)):
        _zero_shared(zq, rows[0], acc, base_n, NB)
        plsc.subcore_barrier()
        _segsum_round(c, xa, xb, zq, sidx_all, didx_all, rows, acc,
                      semg, sems, ECH)
        plsc.subcore_barrier()

        @pl.when(c == 0)
        def _():
            _copy_out_shared(acc, base_n, NB, rows[0], oa)

        @pl.when(c == 1)
        def _():
            _copy_out_shared(acc, base_n, NB, rows[0], ob)


_msg64 = pl.kernel(
    _msg64_body,
    out_type=tuple(jax.ShapeDtypeStruct((NP, 2 * MQ), jnp.float32)
                   for _ in range(4)),
    mesh=_mesh,
    compiler_params=pltpu.CompilerParams(use_tc_tiling_on_sc=False),
    scratch_types=[
        pltpu.VMEM((ECH, C), jnp.int32),
        pltpu.VMEM((ECH, C), jnp.int32),
    ] + [pltpu.VMEM((C, 2 * MQ), jnp.float32) for _ in range(8)] + [
        pltpu.VMEM_SHARED((NP, 2 * MQ), jnp.float32),
        pltpu.SemaphoreType.DMA,
        pltpu.SemaphoreType.DMA,
    ],
)


B = 640
GRID = NP // B

_eighth_spec = pl.BlockSpec((B, MQ), lambda i: (i, 0))


def _tc1_body(we0_ref, we1_ref, we2_ref, we3_ref, we4_ref, we5_ref,
              we6_ref, we7_ref, scal_ref, dsrc_ref, w_ref, *o_refs):
    h = jnp.concatenate([we0_ref[...], we1_ref[...], we2_ref[...],
                         we3_ref[...], we4_ref[...], we5_ref[...],
                         we6_ref[...], we7_ref[...]], axis=1)
    ml = scal_ref[:, 0:1]
    f = scal_ref[:, 1:2]
    lf = scal_ref[:, 2:3]
    ll = scal_ref[:, 3:4]
    col = lax.broadcasted_iota(jnp.int32, (B, D), 1)
    h = h / ml
    h = jnp.where(col == D - 3, f, h)
    h = jnp.where(col == D - 2, lf, h)
    h = jnp.where(col == D - 1, ll, h)
    sout = lax.rsqrt(jnp.maximum(dsrc_ref[:, 0:1], 1.0))
    x = jnp.dot(h * sout, w_ref[...], preferred_element_type=jnp.float32)
    for j, o_ref in enumerate(o_refs):
        o_ref[...] = x[:, j * 2 * MQ:(j + 1) * 2 * MQ]


_quarter_spec = pl.BlockSpec((B, 2 * MQ), lambda i: (i, 0))

_tc1 = pl.pallas_call(
    _tc1_body,
    grid=(GRID,),
    in_specs=[_eighth_spec] * 8 + [
        pl.BlockSpec((B, 4), lambda i: (i, 0)),
        _eighth_spec,
        pl.BlockSpec((D, D), lambda i: (0, 0)),
    ],
    out_specs=[_quarter_spec] * 4,
    out_shape=[jax.ShapeDtypeStruct((NP, 2 * MQ), jnp.float32)] * 4,
)


def _tc2_body(m0_ref, m1_ref, m2_ref, m3_ref,
              dsrc_ref, ddst_ref, b_ref, w_ref, *o_refs):
    agg = jnp.concatenate([m0_ref[...], m1_ref[...],
                           m2_ref[...], m3_ref[...]], axis=1)
    sin = lax.rsqrt(jnp.maximum(ddst_ref[:, 0:1], 1.0))
    sout = lax.rsqrt(jnp.maximum(dsrc_ref[:, 0:1], 1.0))
    h = jnp.maximum(agg * sin + b_ref[...], 0.0)
    x = jnp.dot(h * sout, w_ref[...], preferred_element_type=jnp.float32)
    for j, o_ref in enumerate(o_refs):
        o_ref[...] = x[:, j * MQ:(j + 1) * MQ]


_tc2 = pl.pallas_call(
    _tc2_body,
    grid=(GRID,),
    in_specs=[_quarter_spec] * 4 + [
        _eighth_spec,
        _eighth_spec,
        pl.BlockSpec((1, D), lambda i: (0, 0)),
        pl.BlockSpec((D, D), lambda i: (0, 0)),
    ],
    out_specs=[_eighth_spec] * 8,
    out_shape=[jax.ShapeDtypeStruct((NP, MQ), jnp.float32)] * 8,
)


def _tc3_body(m0_ref, m1_ref, m2_ref, m3_ref, m4_ref, m5_ref, m6_ref,
              m7_ref, ddst_ref, b_ref, oh_ref, og_ref):
    agg = jnp.concatenate([m0_ref[...], m1_ref[...], m2_ref[...],
                           m3_ref[...], m4_ref[...], m5_ref[...],
                           m6_ref[...], m7_ref[...]], axis=1)
    sin = lax.rsqrt(jnp.maximum(ddst_ref[:, 0:1], 1.0))
    h = jnp.maximum(agg * sin + b_ref[...], 0.0)
    oh_ref[...] = h
    i = pl.program_id(0)
    rows = lax.broadcasted_iota(jnp.int32, (B, 1), 0) + i * B
    part = jnp.sum(jnp.where(rows < N, h, 0.0), axis=0, keepdims=True)

    @pl.when(i == 0)
    def _():
        og_ref[...] = jnp.zeros_like(og_ref)

    og_ref[...] += part

    @pl.when(i == GRID - 1)
    def _():
        og_ref[...] = og_ref[...] * (1.0 / N)


_tc3 = pl.pallas_call(
    _tc3_body,
    grid=(GRID,),
    in_specs=[_eighth_spec] * 8 + [
        _eighth_spec,
        pl.BlockSpec((1, D), lambda i: (0, 0)),
    ],
    out_specs=[
        pl.BlockSpec((B, D), lambda i: (i, 0)),
        pl.BlockSpec((1, D), lambda i: (0, 0)),
    ],
    out_shape=[
        jax.ShapeDtypeStruct((NP, D), jnp.float32),
        jax.ShapeDtypeStruct((1, D), jnp.float32),
    ],
)


@jax.jit
def kernel(word_ids, ml, f, lf, ll, edge_index, emb_table, W1, b1, W2, b2):
    t = jnp.pad(emb_table, ((0, 0), (0, D - emb_table.shape[1])))
    tq = [t[:, i * MQ:(i + 1) * MQ] for i in range(NG)]
    wt = jnp.concatenate(
        [word_ids.astype(jnp.int32).reshape(-1),
         jnp.ones(((NP - N) * L,), jnp.int32)]).reshape(16, NB * L)
    wids3 = jnp.concatenate(
        [wt, jnp.ones((16, WPT - NB * L), jnp.int32)], axis=1)
    wids3 = wids3.reshape(16, WCH, C)
    nt = (jnp.arange(NP * L, dtype=jnp.int32) // L).reshape(16, NB * L)
    nids3 = jnp.concatenate(
        [nt, jnp.full((16, WPT - NB * L), TRASH, jnp.int32)], axis=1)
    nids3 = nids3.reshape(16, WCH, C)
    ei = edge_index.astype(jnp.int32)
    pad_e = jnp.full((EP - E,), TRASH, jnp.int32)
    esrc3 = jnp.concatenate([ei[0], pad_e]).reshape(16, ECH, C)
    edst3 = jnp.concatenate([ei[1], pad_e]).reshape(16, ECH, C)
    scal = jnp.stack([ml, f, lf, ll], axis=1)
    scal = jnp.concatenate(
        [scal,
         jnp.concatenate([jnp.ones((NP - N, 1), jnp.float32),
                          jnp.zeros((NP - N, 3), jnp.float32)], axis=1)])
    zm = jnp.zeros((C, MQ), jnp.float32)
    ones32 = jnp.ones((C, MQ), jnp.float32)

    o = _embdeg(*tq, wids3, nids3, esrc3, edst3, zm, ones32)
    we = o[:8]
    dsrc, ddst = o[8], o[9]
    zq = jnp.zeros((C, 2 * MQ), jnp.float32)
    x = _tc1(*we, scal, dsrc, W1)
    m = _msg64(*x, esrc3, edst3, zq)
    y = _tc2(*m, dsrc, ddst, b1.reshape(1, D), W2)
    n = _msg(*y, esrc3, edst3, zm)
    h, hg = _tc3(*n, ddst, b2.reshape(1, D))
    return h[:N], hg


# speedup vs baseline: 1.6722x; 1.5617x over previous
import jax
import jax.numpy as jnp
from jax import lax
from jax.experimental import pallas as pl
from jax.experimental.pallas import tpu as pltpu
from jax.experimental.pallas import tpu_sc as plsc

N = 10000
E = 160000
L = 20
V = 50000
D = 256
Q = 64

NP = 10240
NB = NP // 16
NW = NP // 32
C = 128
TRASH = NP - 1

ECH = 80
EPT = ECH * C
EP = EPT * 16

G4 = 4

_mesh = plsc.VectorSubcoreMesh(core_axis_name="c", subcore_axis_name="s")


def _zero_shared(zhbm, stage, shared, base, nrows):
    pltpu.sync_copy(zhbm, stage)
    for j in range(nrows // C):
        pltpu.sync_copy(stage, shared.at[pl.ds(base + j * C, C)])


def _copy_out_shared(shared, base, nrows, stage, out):
    for j in range(nrows // C):
        pltpu.sync_copy(shared.at[pl.ds(base + j * C, C)], stage)
        pltpu.sync_copy(stage, out.at[pl.ds(base + j * C, C)])


def _segsum_round(c, src_a, src_b, zq, sidx_all, didx_all, rows, acc,
                  semg, sems, nch):
    def group(g, p):
        @pl.when(g >= 2)
        def _():
            for j in range(G4):
                pltpu.make_async_copy(zq, rows[p * G4 + j], sems).wait()

        for j in range(G4):
            k = g * G4 + j

            @pl.when(c == 0)
            def _():
                pltpu.async_copy(src_a.at[sidx_all.at[k]],
                                 rows[p * G4 + j], semg)

            @pl.when(c == 1)
            def _():
                pltpu.async_copy(src_b.at[sidx_all.at[k]],
                                 rows[p * G4 + j], semg)

        for j in range(G4):
            pltpu.make_async_copy(zq, rows[p * G4 + j], semg).wait()
        for j in range(G4):
            k = g * G4 + j
            pltpu.async_copy(rows[p * G4 + j], acc.at[didx_all.at[k]],
                             sems, add=True)

    def body(i, carry):
        group(2 * i, 0)
        group(2 * i + 1, 1)
        return carry

    lax.fori_loop(0, nch // (2 * G4), body, 0)
    for j in range(2 * G4):
        pltpu.make_async_copy(zq, rows[j], sems).wait()


def _embdeg_body(tbl, wids3, esrc3, edst3, z20, z16, ones16,
                 we, dsrc, ddst,
                 widx_all, eidx_all, g0, g1, g2, g3, ovbuf, obuf, st16,
                 dacc, semg, semd):
    c = lax.axis_index("c")
    s = lax.axis_index("s")
    w = c * 16 + s
    base_n = w * NW
    gbuf = (g0, g1, g2, g3)
    pltpu.sync_copy(wids3.at[w], widx_all)

    for j in range(G4):
        pltpu.async_copy(tbl.at[widx_all.at[j]], gbuf[j], semg)

    def node_group(i, carry):
        for j in range(G4):
            n = i * G4 + j
            pltpu.make_async_copy(z20, gbuf[j], semg).wait()
            for v in range(D // 16):
                acc = gbuf[j][0, pl.ds(16 * v, 16)]
                for t in range(1, L):
                    acc = acc + gbuf[j][t, pl.ds(16 * v, 16)]
                ovbuf[j, pl.ds(16 * v, 16)] = acc

            @pl.when(i < NW // G4 - 1)
            def _():
                pltpu.async_copy(tbl.at[widx_all.at[n + G4]], gbuf[j], semg)

        pltpu.sync_copy(ovbuf, we.at[pl.ds(base_n + i * G4, G4)])
        return carry

    lax.fori_loop(0, NW // G4, node_group, 0)

    @pl.when(c == 0)
    def _():
        pltpu.sync_copy(esrc3.at[s], eidx_all)

    @pl.when(c == 1)
    def _():
        pltpu.sync_copy(edst3.at[s], eidx_all)

    pltpu.sync_copy(ones16, obuf)
    _zero_shared(z16, st16, dacc, s * NB, NB)
    plsc.subcore_barrier()

    def deg_step(i, carry):
        for j in range(G4):
            k = i * G4 + j
            pltpu.async_copy(obuf, dacc.at[eidx_all.at[k]], semd, add=True)

        @pl.when(i >= 1)
        def _():
            for j in range(G4):
                pltpu.make_async_copy(z16, obuf, semd).wait()

        return carry

    lax.fori_loop(0, ECH // G4, deg_step, 0)
    for j in range(G4):
        pltpu.make_async_copy(z16, obuf, semd).wait()

    plsc.subcore_barrier()

    @pl.when(c == 0)
    def _():
        _copy_out_shared(dacc, s * NB, NB, st16, dsrc)

    @pl.when(c == 1)
    def _():
        _copy_out_shared(dacc, s * NB, NB, st16, ddst)


_embdeg = pl.kernel(
    _embdeg_body,
    out_type=(
        jax.ShapeDtypeStruct((NP, D), jnp.float32),
        jax.ShapeDtypeStruct((NP, 16), jnp.float32),
        jax.ShapeDtypeStruct((NP, 16), jnp.float32),
    ),
    mesh=_mesh,
    compiler_params=pltpu.CompilerParams(use_tc_tiling_on_sc=False),
    scratch_types=[
        pltpu.VMEM((NW, L), jnp.int32),
        pltpu.VMEM((ECH, C), jnp.int32),
        pltpu.VMEM((L, D), jnp.float32),
        pltpu.VMEM((L, D), jnp.float32),
        pltpu.VMEM((L, D), jnp.float32),
        pltpu.VMEM((L, D), jnp.float32),
        pltpu.VMEM((G4, D), jnp.float32),
        pltpu.VMEM((C, 16), jnp.float32),
        pltpu.VMEM((C, 16), jnp.float32),
        pltpu.VMEM_SHARED((NP, 16), jnp.float32),
        pltpu.SemaphoreType.DMA,
        pltpu.SemaphoreType.DMA,
    ],
)


def _msg_body(x0, x1, x2, x3, esrc3, edst3, zq,
              m0, m1, m2, m3,
              sidx_all, didx_all,
              r0, r1, r2, r3, r4, r5, r6, r7,
              acc, semg, sems):
    c = lax.axis_index("c")
    s = lax.axis_index("s")
    base_n = s * NB
    rows = (r0, r1, r2, r3, r4, r5, r6, r7)
    pltpu.sync_copy(esrc3.at[s], sidx_all)
    pltpu.sync_copy(edst3.at[s], didx_all)

    for xa, xb, oa, ob in ((x0, x2, m0, m2), (x1, x3, m1, m3)):
        _zero_shared(zq, rows[0], acc, base_n, NB)
        plsc.subcore_barrier()
        _segsum_round(c, xa, xb, zq, sidx_all, didx_all, rows, acc,
                      semg, sems, ECH)
        plsc.subcore_barrier()

        @pl.when(c == 0)
        def _():
            _copy_out_shared(acc, base_n, NB, rows[0], oa)

        @pl.when(c == 1)
        def _():
            _copy_out_shared(acc, base_n, NB, rows[0], ob)


_msg = pl.kernel(
    _msg_body,
    out_type=tuple(jax.ShapeDtypeStruct((NP, Q), jnp.float32)
                   for _ in range(4)),
    mesh=_mesh,
    compiler_params=pltpu.CompilerParams(use_tc_tiling_on_sc=False),
    scratch_types=[
        pltpu.VMEM((ECH, C), jnp.int32),
        pltpu.VMEM((ECH, C), jnp.int32),
    ] + [pltpu.VMEM((C, Q), jnp.float32) for _ in range(8)] + [
        pltpu.VMEM_SHARED((NP, Q), jnp.float32),
        pltpu.SemaphoreType.DMA,
        pltpu.SemaphoreType.DMA,
    ],
)

B = 640
GRID = NP // B

_quarter_spec = pl.BlockSpec((B, Q), lambda i: (i, 0))
_deg_spec = pl.BlockSpec((B, 16), lambda i: (i, 0))


def _tc1_body(we_ref, scal_ref, dsrc_ref, w_ref, *o_refs):
    h = we_ref[...]
    ml = scal_ref[:, 0:1]
    f = scal_ref[:, 1:2]
    lf = scal_ref[:, 2:3]
    ll = scal_ref[:, 3:4]
    col = lax.broadcasted_iota(jnp.int32, (B, D), 1)
    h = h / ml
    h = jnp.where(col == D - 3, f, h)
    h = jnp.where(col == D - 2, lf, h)
    h = jnp.where(col == D - 1, ll, h)
    sout = lax.rsqrt(jnp.maximum(dsrc_ref[:, 0:1], 1.0))
    x = jnp.dot(h * sout, w_ref[...], preferred_element_type=jnp.float32)
    for j, o_ref in enumerate(o_refs):
        o_ref[...] = x[:, j * Q:(j + 1) * Q]


_tc1 = pl.pallas_call(
    _tc1_body,
    grid=(GRID,),
    in_specs=[
        pl.BlockSpec((B, D), lambda i: (i, 0)),
        pl.BlockSpec((B, 4), lambda i: (i, 0)),
        _deg_spec,
        pl.BlockSpec((D, D), lambda i: (0, 0)),
    ],
    out_specs=[_quarter_spec] * 4,
    out_shape=[jax.ShapeDtypeStruct((NP, Q), jnp.float32)] * 4,
)


def _tc2_body(m0_ref, m1_ref, m2_ref, m3_ref,
              dsrc_ref, ddst_ref, b_ref, w_ref, *o_refs):
    agg = jnp.concatenate([m0_ref[...], m1_ref[...],
                           m2_ref[...], m3_ref[...]], axis=1)
    sin = lax.rsqrt(jnp.maximum(ddst_ref[:, 0:1], 1.0))
    sout = lax.rsqrt(jnp.maximum(dsrc_ref[:, 0:1], 1.0))
    h = jnp.maximum(agg * sin + b_ref[...], 0.0)
    x = jnp.dot(h * sout, w_ref[...], preferred_element_type=jnp.float32)
    for j, o_ref in enumerate(o_refs):
        o_ref[...] = x[:, j * Q:(j + 1) * Q]


_tc2 = pl.pallas_call(
    _tc2_body,
    grid=(GRID,),
    in_specs=[_quarter_spec] * 4 + [
        _deg_spec,
        _deg_spec,
        pl.BlockSpec((1, D), lambda i: (0, 0)),
        pl.BlockSpec((D, D), lambda i: (0, 0)),
    ],
    out_specs=[_quarter_spec] * 4,
    out_shape=[jax.ShapeDtypeStruct((NP, Q), jnp.float32)] * 4,
)


def _tc3_body(m0_ref, m1_ref, m2_ref, m3_ref, ddst_ref, b_ref,
              oh_ref, og_ref):
    agg = jnp.concatenate([m0_ref[...], m1_ref[...],
                           m2_ref[...], m3_ref[...]], axis=1)
    sin = lax.rsqrt(jnp.maximum(ddst_ref[:, 0:1], 1.0))
    h = jnp.maximum(agg * sin + b_ref[...], 0.0)
    oh_ref[...] = h
    i = pl.program_id(0)
    rows = lax.broadcasted_iota(jnp.int32, (B, 1), 0) + i * B
    part = jnp.sum(jnp.where(rows < N, h, 0.0), axis=0, keepdims=True)

    @pl.when(i == 0)
    def _():
        og_ref[...] = jnp.zeros_like(og_ref)

    og_ref[...] += part

    @pl.when(i == GRID - 1)
    def _():
        og_ref[...] = og_ref[...] * (1.0 / N)


_tc3 = pl.pallas_call(
    _tc3_body,
    grid=(GRID,),
    in_specs=[_quarter_spec] * 4 + [
        _deg_spec,
        pl.BlockSpec((1, D), lambda i: (0, 0)),
    ],
    out_specs=[
        pl.BlockSpec((B, D), lambda i: (i, 0)),
        pl.BlockSpec((1, D), lambda i: (0, 0)),
    ],
    out_shape=[
        jax.ShapeDtypeStruct((NP, D), jnp.float32),
        jax.ShapeDtypeStruct((1, D), jnp.float32),
    ],
)


@jax.jit
def kernel(word_ids, ml, f, lf, ll, edge_index, emb_table, W1, b1, W2, b2):
    tbl = jnp.pad(emb_table, ((0, 0), (0, D - emb_table.shape[1])))
    wids3 = jnp.concatenate(
        [word_ids.astype(jnp.int32).reshape(-1),
         jnp.ones(((NP - N) * L,), jnp.int32)]).reshape(32, NW, L)
    ei = edge_index.astype(jnp.int32)
    pad_e = jnp.full((EP - E,), TRASH, jnp.int32)
    esrc3 = jnp.concatenate([ei[0], pad_e]).reshape(16, ECH, C)
    edst3 = jnp.concatenate([ei[1], pad_e]).reshape(16, ECH, C)
    scal = jnp.stack([ml, f, lf, ll], axis=1)
    scal = jnp.concatenate(
        [scal,
         jnp.concatenate([jnp.ones((NP - N, 1), jnp.float32),
                          jnp.zeros((NP - N, 3), jnp.float32)], axis=1)])
    z20 = jnp.zeros((L, D), jnp.float32)
    z16 = jnp.zeros((C, 16), jnp.float32)
    ones16 = jnp.ones((C, 16), jnp.float32)
    zq = jnp.zeros((C, Q), jnp.float32)

    we, dsrc, ddst = _embdeg(tbl, wids3, esrc3, edst3, z20, z16, ones16)
    x = _tc1(we, scal, dsrc, W1)
    m = _msg(*x, esrc3, edst3, zq)
    y = _tc2(*m, dsrc, ddst, b1.reshape(1, D), W2)
    n = _msg(*y, esrc3, edst3, zq)
    h, hg = _tc3(*n, ddst, b2.reshape(1, D))
    return h[:N], hg


# speedup vs baseline: 1.6822x; 1.0060x over previous
import jax
import jax.numpy as jnp
from jax import lax
from jax.experimental import pallas as pl
from jax.experimental.pallas import tpu as pltpu
from jax.experimental.pallas import tpu_sc as plsc

N = 10000
E = 160000
L = 20
V = 50000
D = 256
Q = 64

NP = 10240
NB = NP // 16
NW = NP // 32
C = 128
TRASH = NP - 1

ECH = 80
EPT = ECH * C
EP = EPT * 16

G4 = 4

_mesh = plsc.VectorSubcoreMesh(core_axis_name="c", subcore_axis_name="s")


def _zero_shared(zhbm, stage, shared, base, nrows):
    pltpu.sync_copy(zhbm, stage)
    for j in range(nrows // C):
        pltpu.sync_copy(stage, shared.at[pl.ds(base + j * C, C)])


def _copy_out_shared(shared, base, nrows, stage, out):
    for j in range(nrows // C):
        pltpu.sync_copy(shared.at[pl.ds(base + j * C, C)], stage)
        pltpu.sync_copy(stage, out.at[pl.ds(base + j * C, C)])


def _segsum_round(c, src_a, src_b, zq, sidx_all, didx_all, rows, acc,
                  semg, sems, nch):
    def group(g, p):
        @pl.when(g >= 2)
        def _():
            for j in range(G4):
                pltpu.make_async_copy(zq, rows[p * G4 + j], sems).wait()

        for j in range(G4):
            k = g * G4 + j

            @pl.when(c == 0)
            def _():
                pltpu.async_copy(src_a.at[sidx_all.at[k]],
                                 rows[p * G4 + j], semg)

            @pl.when(c == 1)
            def _():
                pltpu.async_copy(src_b.at[sidx_all.at[k]],
                                 rows[p * G4 + j], semg)

        for j in range(G4):
            pltpu.make_async_copy(zq, rows[p * G4 + j], semg).wait()
        for j in range(G4):
            k = g * G4 + j
            pltpu.async_copy(rows[p * G4 + j], acc.at[didx_all.at[k]],
                             sems, add=True)

    def body(i, carry):
        group(2 * i, 0)
        group(2 * i + 1, 1)
        return carry

    lax.fori_loop(0, nch // (2 * G4), body, 0)
    for j in range(2 * G4):
        pltpu.make_async_copy(zq, rows[j], sems).wait()


def _embdeg_body(tbl, wids3, esrc3, edst3, z20, z16, ones16,
                 we, dsrc, ddst,
                 widx_all, eidx_all, g0, g1, g2, g3, ovbuf, obuf, st16,
                 dacc, semg, semd):
    c = lax.axis_index("c")
    s = lax.axis_index("s")
    w = c * 16 + s
    base_n = w * NW
    gbuf = (g0, g1, g2, g3)
    pltpu.sync_copy(wids3.at[w], widx_all)

    @pl.when(c == 0)
    def _():
        pltpu.sync_copy(esrc3.at[s], eidx_all)

    @pl.when(c == 1)
    def _():
        pltpu.sync_copy(edst3.at[s], eidx_all)

    pltpu.sync_copy(ones16, obuf)
    _zero_shared(z16, st16, dacc, s * NB, NB)
    plsc.subcore_barrier()

    for j in range(G4):
        pltpu.async_copy(tbl.at[widx_all.at[j]], gbuf[j], semg)

    def node_group(i, carry):
        pltpu.async_copy(obuf, dacc.at[eidx_all.at[i]], semd, add=True)
        for j in range(G4):
            n = i * G4 + j
            pltpu.make_async_copy(z20, gbuf[j], semg).wait()
            for v in range(D // 16):
                acc = gbuf[j][0, pl.ds(16 * v, 16)]
                for t in range(1, L):
                    acc = acc + gbuf[j][t, pl.ds(16 * v, 16)]
                ovbuf[j, pl.ds(16 * v, 16)] = acc

            @pl.when(i < NW // G4 - 1)
            def _():
                pltpu.async_copy(tbl.at[widx_all.at[n + G4]], gbuf[j], semg)

        pltpu.sync_copy(ovbuf, we.at[pl.ds(base_n + i * G4, G4)])

        @pl.when(i >= 1)
        def _():
            pltpu.make_async_copy(z16, obuf, semd).wait()

        return carry

    lax.fori_loop(0, NW // G4, node_group, 0)
    pltpu.make_async_copy(z16, obuf, semd).wait()

    plsc.subcore_barrier()

    @pl.when(c == 0)
    def _():
        _copy_out_shared(dacc, s * NB, NB, st16, dsrc)

    @pl.when(c == 1)
    def _():
        _copy_out_shared(dacc, s * NB, NB, st16, ddst)


_embdeg = pl.kernel(
    _embdeg_body,
    out_type=(
        jax.ShapeDtypeStruct((NP, D), jnp.float32),
        jax.ShapeDtypeStruct((NP, 16), jnp.float32),
        jax.ShapeDtypeStruct((NP, 16), jnp.float32),
    ),
    mesh=_mesh,
    compiler_params=pltpu.CompilerParams(use_tc_tiling_on_sc=False),
    scratch_types=[
        pltpu.VMEM((NW, L), jnp.int32),
        pltpu.VMEM((ECH, C), jnp.int32),
        pltpu.VMEM((L, D), jnp.float32),
        pltpu.VMEM((L, D), jnp.float32),
        pltpu.VMEM((L, D), jnp.float32),
        pltpu.VMEM((L, D), jnp.float32),
        pltpu.VMEM((G4, D), jnp.float32),
        pltpu.VMEM((C, 16), jnp.float32),
        pltpu.VMEM((C, 16), jnp.float32),
        pltpu.VMEM_SHARED((NP, 16), jnp.float32),
        pltpu.SemaphoreType.DMA,
        pltpu.SemaphoreType.DMA,
    ],
)


def _msg_body(x0, x1, x2, x3, esrc3, edst3, zq,
              m0, m1, m2, m3,
              sidx_all, didx_all,
              r0, r1, r2, r3, r4, r5, r6, r7,
              acc, semg, sems):
    c = lax.axis_index("c")
    s = lax.axis_index("s")
    base_n = s * NB
    rows = (r0, r1, r2, r3, r4, r5, r6, r7)
    pltpu.sync_copy(esrc3.at[s], sidx_all)
    pltpu.sync_copy(edst3.at[s], didx_all)

    for xa, xb, oa, ob in ((x0, x2, m0, m2), (x1, x3, m1, m3)):
        _zero_shared(zq, rows[0], acc, base_n, NB)
        plsc.subcore_barrier()
        _segsum_round(c, xa, xb, zq, sidx_all, didx_all, rows, acc,
                      semg, sems, ECH)
        plsc.subcore_barrier()

        @pl.when(c == 0)
        def _():
            _copy_out_shared(acc, base_n, NB, rows[0], oa)

        @pl.when(c == 1)
        def _():
            _copy_out_shared(acc, base_n, NB, rows[0], ob)


_msg = pl.kernel(
    _msg_body,
    out_type=tuple(jax.ShapeDtypeStruct((NP, Q), jnp.float32)
                   for _ in range(4)),
    mesh=_mesh,
    compiler_params=pltpu.CompilerParams(use_tc_tiling_on_sc=False),
    scratch_types=[
        pltpu.VMEM((ECH, C), jnp.int32),
        pltpu.VMEM((ECH, C), jnp.int32),
    ] + [pltpu.VMEM((C, Q), jnp.float32) for _ in range(8)] + [
        pltpu.VMEM_SHARED((NP, Q), jnp.float32),
        pltpu.SemaphoreType.DMA,
        pltpu.SemaphoreType.DMA,
    ],
)

B = 640
GRID = NP // B

_quarter_spec = pl.BlockSpec((B, Q), lambda i: (i, 0))
_deg_spec = pl.BlockSpec((B, 16), lambda i: (i, 0))


def _tc1_body(we_ref, scal_ref, dsrc_ref, w_ref, *o_refs):
    h = we_ref[...]
    ml = scal_ref[:, 0:1]
    f = scal_ref[:, 1:2]
    lf = scal_ref[:, 2:3]
    ll = scal_ref[:, 3:4]
    col = lax.broadcasted_iota(jnp.int32, (B, D), 1)
    h = h / ml
    h = jnp.where(col == D - 3, f, h)
    h = jnp.where(col == D - 2, lf, h)
    h = jnp.where(col == D - 1, ll, h)
    sout = lax.rsqrt(jnp.maximum(dsrc_ref[:, 0:1], 1.0))
    x = jnp.dot(h * sout, w_ref[...], preferred_element_type=jnp.float32)
    for j, o_ref in enumerate(o_refs):
        o_ref[...] = x[:, j * Q:(j + 1) * Q]


_tc1 = pl.pallas_call(
    _tc1_body,
    grid=(GRID,),
    in_specs=[
        pl.BlockSpec((B, D), lambda i: (i, 0)),
        pl.BlockSpec((B, 4), lambda i: (i, 0)),
        _deg_spec,
        pl.BlockSpec((D, D), lambda i: (0, 0)),
    ],
    out_specs=[_quarter_spec] * 4,
    out_shape=[jax.ShapeDtypeStruct((NP, Q), jnp.float32)] * 4,
)


def _tc2_body(m0_ref, m1_ref, m2_ref, m3_ref,
              dsrc_ref, ddst_ref, b_ref, w_ref, *o_refs):
    agg = jnp.concatenate([m0_ref[...], m1_ref[...],
                           m2_ref[...], m3_ref[...]], axis=1)
    sin = lax.rsqrt(jnp.maximum(ddst_ref[:, 0:1], 1.0))
    sout = lax.rsqrt(jnp.maximum(dsrc_ref[:, 0:1], 1.0))
    h = jnp.maximum(agg * sin + b_ref[...], 0.0)
    x = jnp.dot(h * sout, w_ref[...], preferred_element_type=jnp.float32)
    for j, o_ref in enumerate(o_refs):
        o_ref[...] = x[:, j * Q:(j + 1) * Q]


_tc2 = pl.pallas_call(
    _tc2_body,
    grid=(GRID,),
    in_specs=[_quarter_spec] * 4 + [
        _deg_spec,
        _deg_spec,
        pl.BlockSpec((1, D), lambda i: (0, 0)),
        pl.BlockSpec((D, D), lambda i: (0, 0)),
    ],
    out_specs=[_quarter_spec] * 4,
    out_shape=[jax.ShapeDtypeStruct((NP, Q), jnp.float32)] * 4,
)


def _tc3_body(m0_ref, m1_ref, m2_ref, m3_ref, ddst_ref, b_ref,
              oh_ref, og_ref):
    agg = jnp.concatenate([m0_ref[...], m1_ref[...],
                           m2_ref[...], m3_ref[...]], axis=1)
    sin = lax.rsqrt(jnp.maximum(ddst_ref[:, 0:1], 1.0))
    h = jnp.maximum(agg * sin + b_ref[...], 0.0)
    oh_ref[...] = h
    i = pl.program_id(0)
    rows = lax.broadcasted_iota(jnp.int32, (B, 1), 0) + i * B
    part = jnp.sum(jnp.where(rows < N, h, 0.0), axis=0, keepdims=True)

    @pl.when(i == 0)
    def _():
        og_ref[...] = jnp.zeros_like(og_ref)

    og_ref[...] += part

    @pl.when(i == GRID - 1)
    def _():
        og_ref[...] = og_ref[...] * (1.0 / N)


_tc3 = pl.pallas_call(
    _tc3_body,
    grid=(GRID,),
    in_specs=[_quarter_spec] * 4 + [
        _deg_spec,
        pl.BlockSpec((1, D), lambda i: (0, 0)),
    ],
    out_specs=[
        pl.BlockSpec((B, D), lambda i: (i, 0)),
        pl.BlockSpec((1, D), lambda i: (0, 0)),
    ],
    out_shape=[
        jax.ShapeDtypeStruct((NP, D), jnp.float32),
        jax.ShapeDtypeStruct((1, D), jnp.float32),
    ],
)


@jax.jit
def kernel(word_ids, ml, f, lf, ll, edge_index, emb_table, W1, b1, W2, b2):
    tbl = jnp.pad(emb_table, ((0, 0), (0, D - emb_table.shape[1])))
    wids3 = jnp.concatenate(
        [word_ids.astype(jnp.int32).reshape(-1),
         jnp.ones(((NP - N) * L,), jnp.int32)]).reshape(32, NW, L)
    ei = edge_index.astype(jnp.int32)
    pad_e = jnp.full((EP - E,), TRASH, jnp.int32)
    esrc3 = jnp.concatenate([ei[0], pad_e]).reshape(16, ECH, C)
    edst3 = jnp.concatenate([ei[1], pad_e]).reshape(16, ECH, C)
    scal = jnp.stack([ml, f, lf, ll], axis=1)
    scal = jnp.concatenate(
        [scal,
         jnp.concatenate([jnp.ones((NP - N, 1), jnp.float32),
                          jnp.zeros((NP - N, 3), jnp.float32)], axis=1)])
    z20 = jnp.zeros((L, D), jnp.float32)
    z16 = jnp.zeros((C, 16), jnp.float32)
    ones16 = jnp.ones((C, 16), jnp.float32)
    zq = jnp.zeros((C, Q), jnp.float32)

    we, dsrc, ddst = _embdeg(tbl, wids3, esrc3, edst3, z20, z16, ones16)
    x = _tc1(we, scal, dsrc, W1)
    m = _msg(*x, esrc3, edst3, zq)
    y = _tc2(*m, dsrc, ddst, b1.reshape(1, D), W2)
    n = _msg(*y, esrc3, edst3, zq)
    h, hg = _tc3(*n, ddst, b2.reshape(1, D))
    return h[:N], hg


# speedup vs baseline: 1.6888x; 1.0039x over previous
import jax
import jax.numpy as jnp
from jax import lax
from jax.experimental import pallas as pl
from jax.experimental.pallas import tpu as pltpu
from jax.experimental.pallas import tpu_sc as plsc

N = 10000
E = 160000
L = 20
V = 50000
D = 256
Q = 64

NP = 10240
NB = NP // 16
NW = NP // 32
C = 128
TRASH = NP - 1

ECH = 80
EPT = ECH * C
EP = EPT * 16

G4 = 4

_mesh = plsc.VectorSubcoreMesh(core_axis_name="c", subcore_axis_name="s")


def _zero_shared(zhbm, stage, shared, base, nrows, sem):
    pltpu.sync_copy(zhbm, stage)
    for j in range(nrows // C):
        pltpu.async_copy(stage, shared.at[pl.ds(base + j * C, C)], sem)
    for j in range(nrows // C):
        pltpu.make_async_copy(zhbm, stage, sem).wait()


def _copy_out_shared(shared, base, nrows, stages, out, zhbm, sem):
    nch = nrows // C
    assert nch == len(stages)
    for j in range(nch):
        pltpu.async_copy(shared.at[pl.ds(base + j * C, C)], stages[j], sem)
    for j in range(nch):
        pltpu.make_async_copy(zhbm, stages[j], sem).wait()
    for j in range(nch):
        pltpu.async_copy(stages[j], out.at[pl.ds(base + j * C, C)], sem)
    for j in range(nch):
        pltpu.make_async_copy(zhbm, stages[j], sem).wait()


def _segsum_round(c, src_a, src_b, zq, sidx_all, didx_all, rows, acc,
                  semg, sems, nch):
    def group(g, p):
        @pl.when(g >= 2)
        def _():
            for j in range(G4):
                pltpu.make_async_copy(zq, rows[p * G4 + j], sems).wait()

        for j in range(G4):
            k = g * G4 + j

            @pl.when(c == 0)
            def _():
                pltpu.async_copy(src_a.at[sidx_all.at[k]],
                                 rows[p * G4 + j], semg)

            @pl.when(c == 1)
            def _():
                pltpu.async_copy(src_b.at[sidx_all.at[k]],
                                 rows[p * G4 + j], semg)

        for j in range(G4):
            pltpu.make_async_copy(zq, rows[p * G4 + j], semg).wait()
        for j in range(G4):
            k = g * G4 + j
            pltpu.async_copy(rows[p * G4 + j], acc.at[didx_all.at[k]],
                             sems, add=True)

    def body(i, carry):
        group(2 * i, 0)
        group(2 * i + 1, 1)
        return carry

    lax.fori_loop(0, nch // (2 * G4), body, 0)
    for j in range(2 * G4):
        pltpu.make_async_copy(zq, rows[j], sems).wait()


def _embdeg_body(tbl, wids3, esrc3, edst3, z20, z16, ones16,
                 we, dsrc, ddst,
                 widx_all, eidx_all, g0, g1, g2, g3, ovbuf, obuf,
                 s16a, s16b, s16c, s16d, s16e,
                 dacc, semg, semd):
    st16 = (s16a, s16b, s16c, s16d, s16e)
    c = lax.axis_index("c")
    s = lax.axis_index("s")
    w = c * 16 + s
    base_n = w * NW
    gbuf = (g0, g1, g2, g3)
    pltpu.sync_copy(wids3.at[w], widx_all)

    @pl.when(c == 0)
    def _():
        pltpu.sync_copy(esrc3.at[s], eidx_all)

    @pl.when(c == 1)
    def _():
        pltpu.sync_copy(edst3.at[s], eidx_all)

    pltpu.sync_copy(ones16, obuf)
    _zero_shared(z16, st16[0], dacc, s * NB, NB, semd)
    plsc.subcore_barrier()

    for j in range(G4):
        pltpu.async_copy(tbl.at[widx_all.at[j]], gbuf[j], semg)

    def node_group(i, carry):
        pltpu.async_copy(obuf, dacc.at[eidx_all.at[i]], semd, add=True)
        for j in range(G4):
            n = i * G4 + j
            pltpu.make_async_copy(z20, gbuf[j], semg).wait()
            for v in range(D // 16):
                acc = gbuf[j][0, pl.ds(16 * v, 16)]
                for t in range(1, L):
                    acc = acc + gbuf[j][t, pl.ds(16 * v, 16)]
                ovbuf[j, pl.ds(16 * v, 16)] = acc

            @pl.when(i < NW // G4 - 1)
            def _():
                pltpu.async_copy(tbl.at[widx_all.at[n + G4]], gbuf[j], semg)

        pltpu.sync_copy(ovbuf, we.at[pl.ds(base_n + i * G4, G4)])

        @pl.when(i >= 1)
        def _():
            pltpu.make_async_copy(z16, obuf, semd).wait()

        return carry

    lax.fori_loop(0, NW // G4, node_group, 0)
    pltpu.make_async_copy(z16, obuf, semd).wait()

    plsc.subcore_barrier()

    @pl.when(c == 0)
    def _():
        _copy_out_shared(dacc, s * NB, NB, st16, dsrc, z16, semd)

    @pl.when(c == 1)
    def _():
        _copy_out_shared(dacc, s * NB, NB, st16, ddst, z16, semd)


_embdeg = pl.kernel(
    _embdeg_body,
    out_type=(
        jax.ShapeDtypeStruct((NP, D), jnp.float32),
        jax.ShapeDtypeStruct((NP, 16), jnp.float32),
        jax.ShapeDtypeStruct((NP, 16), jnp.float32),
    ),
    mesh=_mesh,
    compiler_params=pltpu.CompilerParams(use_tc_tiling_on_sc=False),
    scratch_types=[
        pltpu.VMEM((NW, L), jnp.int32),
        pltpu.VMEM((ECH, C), jnp.int32),
        pltpu.VMEM((L, D), jnp.float32),
        pltpu.VMEM((L, D), jnp.float32),
        pltpu.VMEM((L, D), jnp.float32),
        pltpu.VMEM((L, D), jnp.float32),
        pltpu.VMEM((G4, D), jnp.float32),
        pltpu.VMEM((C, 16), jnp.float32),
    ] + [pltpu.VMEM((C, 16), jnp.float32) for _ in range(5)] + [
        pltpu.VMEM_SHARED((NP, 16), jnp.float32),
        pltpu.SemaphoreType.DMA,
        pltpu.SemaphoreType.DMA,
    ],
)


def _msg_body(x0, x1, x2, x3, esrc3, edst3, zq,
              m0, m1, m2, m3,
              sidx_all, didx_all,
              r0, r1, r2, r3, r4, r5, r6, r7,
              acc, semg, sems):
    c = lax.axis_index("c")
    s = lax.axis_index("s")
    base_n = s * NB
    rows = (r0, r1, r2, r3, r4, r5, r6, r7)
    pltpu.sync_copy(esrc3.at[s], sidx_all)
    pltpu.sync_copy(edst3.at[s], didx_all)

    for xa, xb, oa, ob in ((x0, x2, m0, m2), (x1, x3, m1, m3)):
        _zero_shared(zq, rows[0], acc, base_n, NB, sems)
        plsc.subcore_barrier()
        _segsum_round(c, xa, xb, zq, sidx_all, didx_all, rows, acc,
                      semg, sems, ECH)
        plsc.subcore_barrier()

        @pl.when(c == 0)
        def _():
            _copy_out_shared(acc, base_n, NB, rows[:5], oa, zq, sems)

        @pl.when(c == 1)
        def _():
            _copy_out_shared(acc, base_n, NB, rows[:5], ob, zq, sems)


_msg = pl.kernel(
    _msg_body,
    out_type=tuple(jax.ShapeDtypeStruct((NP, Q), jnp.float32)
                   for _ in range(4)),
    mesh=_mesh,
    compiler_params=pltpu.CompilerParams(use_tc_tiling_on_sc=False),
    scratch_types=[
        pltpu.VMEM((ECH, C), jnp.int32),
        pltpu.VMEM((ECH, C), jnp.int32),
    ] + [pltpu.VMEM((C, Q), jnp.float32) for _ in range(8)] + [
        pltpu.VMEM_SHARED((NP, Q), jnp.float32),
        pltpu.SemaphoreType.DMA,
        pltpu.SemaphoreType.DMA,
    ],
)

B = 640
GRID = NP // B

_quarter_spec = pl.BlockSpec((B, Q), lambda i: (i, 0))
_deg_spec = pl.BlockSpec((B, 16), lambda i: (i, 0))


def _tc1_body(we_ref, scal_ref, dsrc_ref, w_ref, *o_refs):
    h = we_ref[...]
    ml = scal_ref[:, 0:1]
    f = scal_ref[:, 1:2]
    lf = scal_ref[:, 2:3]
    ll = scal_ref[:, 3:4]
    col = lax.broadcasted_iota(jnp.int32, (B, D), 1)
    h = h / ml
    h = jnp.where(col == D - 3, f, h)
    h = jnp.where(col == D - 2, lf, h)
    h = jnp.where(col == D - 1, ll, h)
    sout = lax.rsqrt(jnp.maximum(dsrc_ref[:, 0:1], 1.0))
    x = jnp.dot(h * sout, w_ref[...], preferred_element_type=jnp.float32)
    for j, o_ref in enumerate(o_refs):
        o_ref[...] = x[:, j * Q:(j + 1) * Q]


_tc1 = pl.pallas_call(
    _tc1_body,
    grid=(GRID,),
    in_specs=[
        pl.BlockSpec((B, D), lambda i: (i, 0)),
        pl.BlockSpec((B, 4), lambda i: (i, 0)),
        _deg_spec,
        pl.BlockSpec((D, D), lambda i: (0, 0)),
    ],
    out_specs=[_quarter_spec] * 4,
    out_shape=[jax.ShapeDtypeStruct((NP, Q), jnp.float32)] * 4,
)


def _tc2_body(m0_ref, m1_ref, m2_ref, m3_ref,
              dsrc_ref, ddst_ref, b_ref, w_ref, *o_refs):
    agg = jnp.concatenate([m0_ref[...], m1_ref[...],
                           m2_ref[...], m3_ref[...]], axis=1)
    sin = lax.rsqrt(jnp.maximum(ddst_ref[:, 0:1], 1.0))
    sout = lax.rsqrt(jnp.maximum(dsrc_ref[:, 0:1], 1.0))
    h = jnp.maximum(agg * sin + b_ref[...], 0.0)
    x = jnp.dot(h * sout, w_ref[...], preferred_element_type=jnp.float32)
    for j, o_ref in enumerate(o_refs):
        o_ref[...] = x[:, j * Q:(j + 1) * Q]


_tc2 = pl.pallas_call(
    _tc2_body,
    grid=(GRID,),
    in_specs=[_quarter_spec] * 4 + [
        _deg_spec,
        _deg_spec,
        pl.BlockSpec((1, D), lambda i: (0, 0)),
        pl.BlockSpec((D, D), lambda i: (0, 0)),
    ],
    out_specs=[_quarter_spec] * 4,
    out_shape=[jax.ShapeDtypeStruct((NP, Q), jnp.float32)] * 4,
)


def _tc3_body(m0_ref, m1_ref, m2_ref, m3_ref, ddst_ref, b_ref,
              oh_ref, og_ref):
    agg = jnp.concatenate([m0_ref[...], m1_ref[...],
                           m2_ref[...], m3_ref[...]], axis=1)
    sin = lax.rsqrt(jnp.maximum(ddst_ref[:, 0:1], 1.0))
    h = jnp.maximum(agg * sin + b_ref[...], 0.0)
    oh_ref[...] = h
    i = pl.program_id(0)
    rows = lax.broadcasted_iota(jnp.int32, (B, 1), 0) + i * B
    part = jnp.sum(jnp.where(rows < N, h, 0.0), axis=0, keepdims=True)

    @pl.when(i == 0)
    def _():
        og_ref[...] = jnp.zeros_like(og_ref)

    og_ref[...] += part

    @pl.when(i == GRID - 1)
    def _():
        og_ref[...] = og_ref[...] * (1.0 / N)


_tc3 = pl.pallas_call(
    _tc3_body,
    grid=(GRID,),
    in_specs=[_quarter_spec] * 4 + [
        _deg_spec,
        pl.BlockSpec((1, D), lambda i: (0, 0)),
    ],
    out_specs=[
        pl.BlockSpec((B, D), lambda i: (i, 0)),
        pl.BlockSpec((1, D), lambda i: (0, 0)),
    ],
    out_shape=[
        jax.ShapeDtypeStruct((NP, D), jnp.float32),
        jax.ShapeDtypeStruct((1, D), jnp.float32),
    ],
)


@jax.jit
def kernel(word_ids, ml, f, lf, ll, edge_index, emb_table, W1, b1, W2, b2):
    tbl = jnp.pad(emb_table, ((0, 0), (0, D - emb_table.shape[1])))
    wids3 = jnp.concatenate(
        [word_ids.astype(jnp.int32).reshape(-1),
         jnp.ones(((NP - N) * L,), jnp.int32)]).reshape(32, NW, L)
    ei = edge_index.astype(jnp.int32)
    pad_e = jnp.full((EP - E,), TRASH, jnp.int32)
    esrc3 = jnp.concatenate([ei[0], pad_e]).reshape(16, ECH, C)
    edst3 = jnp.concatenate([ei[1], pad_e]).reshape(16, ECH, C)
    scal = jnp.stack([ml, f, lf, ll], axis=1)
    scal = jnp.concatenate(
        [scal,
         jnp.concatenate([jnp.ones((NP - N, 1), jnp.float32),
                          jnp.zeros((NP - N, 3), jnp.float32)], axis=1)])
    z20 = jnp.zeros((L, D), jnp.float32)
    z16 = jnp.zeros((C, 16), jnp.float32)
    ones16 = jnp.ones((C, 16), jnp.float32)
    zq = jnp.zeros((C, Q), jnp.float32)

    we, dsrc, ddst = _embdeg(tbl, wids3, esrc3, edst3, z20, z16, ones16)
    x = _tc1(we, scal, dsrc, W1)
    m = _msg(*x, esrc3, edst3, zq)
    y = _tc2(*m, dsrc, ddst, b1.reshape(1, D), W2)
    n = _msg(*y, esrc3, edst3, zq)
    h, hg = _tc3(*n, ddst, b2.reshape(1, D))
    return h[:N], hg


# speedup vs baseline: 1.6904x; 1.0009x over previous
import jax
import jax.numpy as jnp
from jax import lax
from jax.experimental import pallas as pl
from jax.experimental.pallas import tpu as pltpu
from jax.experimental.pallas import tpu_sc as plsc

N = 10000
E = 160000
L = 20
V = 50000
D = 256
Q = 64

NP = 10240
NB = NP // 16
NW = NP // 32
C = 128
TRASH = NP - 1

ECH = 80
EPT = ECH * C
EP = EPT * 16

G4 = 4

_mesh = plsc.VectorSubcoreMesh(core_axis_name="c", subcore_axis_name="s")


def _zero_shared(zhbm, stage, shared, base, nrows, sem):
    pltpu.sync_copy(zhbm, stage)
    for j in range(nrows // C):
        pltpu.async_copy(stage, shared.at[pl.ds(base + j * C, C)], sem)
    for j in range(nrows // C):
        pltpu.make_async_copy(zhbm, stage, sem).wait()


def _copy_out_shared(shared, base, nrows, stages, out, zhbm, sem):
    nch = nrows // C
    assert nch == len(stages)
    for j in range(nch):
        pltpu.async_copy(shared.at[pl.ds(base + j * C, C)], stages[j], sem)
    for j in range(nch):
        pltpu.make_async_copy(zhbm, stages[j], sem).wait()
    for j in range(nch):
        pltpu.async_copy(stages[j], out.at[pl.ds(base + j * C, C)], sem)
    for j in range(nch):
        pltpu.make_async_copy(zhbm, stages[j], sem).wait()


def _segsum_round(c, src_a, src_b, zq, sidx_all, didx_all, rows, acc,
                  semg, sems, nch):
    def group(g, p):
        @pl.when(g >= 2)
        def _():
            for j in range(G4):
                pltpu.make_async_copy(zq, rows[p * G4 + j], sems).wait()

        for j in range(G4):
            k = g * G4 + j

            @pl.when(c == 0)
            def _():
                pltpu.async_copy(src_a.at[sidx_all.at[k]],
                                 rows[p * G4 + j], semg.at[p * G4 + j])

            @pl.when(c == 1)
            def _():
                pltpu.async_copy(src_b.at[sidx_all.at[k]],
                                 rows[p * G4 + j], semg.at[p * G4 + j])

        for j in range(G4):
            k = g * G4 + j
            pltpu.make_async_copy(zq, rows[p * G4 + j],
                                  semg.at[p * G4 + j]).wait()
            pltpu.async_copy(rows[p * G4 + j], acc.at[didx_all.at[k]],
                             sems, add=True)

    def body(i, carry):
        group(2 * i, 0)
        group(2 * i + 1, 1)
        return carry

    lax.fori_loop(0, nch // (2 * G4), body, 0)
    for j in range(2 * G4):
        pltpu.make_async_copy(zq, rows[j], sems).wait()


def _embdeg_body(tbl, wids3, esrc3, edst3, z20, z16, ones16,
                 we, dsrc, ddst,
                 widx_all, eidx_all, g0, g1, g2, g3, ovbuf, obuf,
                 s16a, s16b, s16c, s16d, s16e,
                 dacc, semg, semd):
    st16 = (s16a, s16b, s16c, s16d, s16e)
    c = lax.axis_index("c")
    s = lax.axis_index("s")
    w = c * 16 + s
    base_n = w * NW
    gbuf = (g0, g1, g2, g3)
    pltpu.sync_copy(wids3.at[w], widx_all)

    @pl.when(c == 0)
    def _():
        pltpu.sync_copy(esrc3.at[s], eidx_all)

    @pl.when(c == 1)
    def _():
        pltpu.sync_copy(edst3.at[s], eidx_all)

    pltpu.sync_copy(ones16, obuf)
    _zero_shared(z16, st16[0], dacc, s * NB, NB, semd)
    plsc.subcore_barrier()

    for j in range(G4):
        pltpu.async_copy(tbl.at[widx_all.at[j]], gbuf[j], semg)

    def node_group(i, carry):
        pltpu.async_copy(obuf, dacc.at[eidx_all.at[i]], semd, add=True)
        for j in range(G4):
            n = i * G4 + j
            pltpu.make_async_copy(z20, gbuf[j], semg).wait()
            for v in range(D // 16):
                acc = gbuf[j][0, pl.ds(16 * v, 16)]
                for t in range(1, L):
                    acc = acc + gbuf[j][t, pl.ds(16 * v, 16)]
                ovbuf[j, pl.ds(16 * v, 16)] = acc

            @pl.when(i < NW // G4 - 1)
            def _():
                pltpu.async_copy(tbl.at[widx_all.at[n + G4]], gbuf[j], semg)

        pltpu.sync_copy(ovbuf, we.at[pl.ds(base_n + i * G4, G4)])

        @pl.when(i >= 1)
        def _():
            pltpu.make_async_copy(z16, obuf, semd).wait()

        return carry

    lax.fori_loop(0, NW // G4, node_group, 0)
    pltpu.make_async_copy(z16, obuf, semd).wait()

    plsc.subcore_barrier()

    @pl.when(c == 0)
    def _():
        _copy_out_shared(dacc, s * NB, NB, st16, dsrc, z16, semd)

    @pl.when(c == 1)
    def _():
        _copy_out_shared(dacc, s * NB, NB, st16, ddst, z16, semd)


_embdeg = pl.kernel(
    _embdeg_body,
    out_type=(
        jax.ShapeDtypeStruct((NP, D), jnp.float32),
        jax.ShapeDtypeStruct((NP, 16), jnp.float32),
        jax.ShapeDtypeStruct((NP, 16), jnp.float32),
    ),
    mesh=_mesh,
    compiler_params=pltpu.CompilerParams(use_tc_tiling_on_sc=False),
    scratch_types=[
        pltpu.VMEM((NW, L), jnp.int32),
        pltpu.VMEM((ECH, C), jnp.int32),
        pltpu.VMEM((L, D), jnp.float32),
        pltpu.VMEM((L, D), jnp.float32),
        pltpu.VMEM((L, D), jnp.float32),
        pltpu.VMEM((L, D), jnp.float32),
        pltpu.VMEM((G4, D), jnp.float32),
        pltpu.VMEM((C, 16), jnp.float32),
    ] + [pltpu.VMEM((C, 16), jnp.float32) for _ in range(5)] + [
        pltpu.VMEM_SHARED((NP, 16), jnp.float32),
        pltpu.SemaphoreType.DMA,
        pltpu.SemaphoreType.DMA,
    ],
)


def _msg_body(x0, x1, x2, x3, esrc3, edst3, zq,
              m0, m1, m2, m3,
              sidx_all, didx_all,
              r0, r1, r2, r3, r4, r5, r6, r7,
              acc, semg, sems):
    c = lax.axis_index("c")
    s = lax.axis_index("s")
    base_n = s * NB
    rows = (r0, r1, r2, r3, r4, r5, r6, r7)
    pltpu.sync_copy(esrc3.at[s], sidx_all)
    pltpu.sync_copy(edst3.at[s], didx_all)

    for xa, xb, oa, ob in ((x0, x2, m0, m2), (x1, x3, m1, m3)):
        _zero_shared(zq, rows[0], acc, base_n, NB, sems)
        plsc.subcore_barrier()
        _segsum_round(c, xa, xb, zq, sidx_all, didx_all, rows, acc,
                      semg, sems, ECH)
        plsc.subcore_barrier()

        @pl.when(c == 0)
        def _():
            _copy_out_shared(acc, base_n, NB, rows[:5], oa, zq, sems)

        @pl.when(c == 1)
        def _():
            _copy_out_shared(acc, base_n, NB, rows[:5], ob, zq, sems)


_msg = pl.kernel(
    _msg_body,
    out_type=tuple(jax.ShapeDtypeStruct((NP, Q), jnp.float32)
                   for _ in range(4)),
    mesh=_mesh,
    compiler_params=pltpu.CompilerParams(use_tc_tiling_on_sc=False),
    scratch_types=[
        pltpu.VMEM((ECH, C), jnp.int32),
        pltpu.VMEM((ECH, C), jnp.int32),
    ] + [pltpu.VMEM((C, Q), jnp.float32) for _ in range(8)] + [
        pltpu.VMEM_SHARED((NP, Q), jnp.float32),
        pltpu.SemaphoreType.DMA((8,)),
        pltpu.SemaphoreType.DMA,
    ],
)

B = 640
GRID = NP // B

_quarter_spec = pl.BlockSpec((B, Q), lambda i: (i, 0))
_deg_spec = pl.BlockSpec((B, 16), lambda i: (i, 0))


def _tc1_body(we_ref, scal_ref, dsrc_ref, w_ref, *o_refs):
    h = we_ref[...]
    ml = scal_ref[:, 0:1]
    f = scal_ref[:, 1:2]
    lf = scal_ref[:, 2:3]
    ll = scal_ref[:, 3:4]
    col = lax.broadcasted_iota(jnp.int32, (B, D), 1)
    h = h / ml
    h = jnp.where(col == D - 3, f, h)
    h = jnp.where(col == D - 2, lf, h)
    h = jnp.where(col == D - 1, ll, h)
    sout = lax.rsqrt(jnp.maximum(dsrc_ref[:, 0:1], 1.0))
    x = jnp.dot(h * sout, w_ref[...], preferred_element_type=jnp.float32)
    for j, o_ref in enumerate(o_refs):
        o_ref[...] = x[:, j * Q:(j + 1) * Q]


_tc1 = pl.pallas_call(
    _tc1_body,
    grid=(GRID,),
    in_specs=[
        pl.BlockSpec((B, D), lambda i: (i, 0)),
        pl.BlockSpec((B, 4), lambda i: (i, 0)),
        _deg_spec,
        pl.BlockSpec((D, D), lambda i: (0, 0)),
    ],
    out_specs=[_quarter_spec] * 4,
    out_shape=[jax.ShapeDtypeStruct((NP, Q), jnp.float32)] * 4,
)


def _tc2_body(m0_ref, m1_ref, m2_ref, m3_ref,
              dsrc_ref, ddst_ref, b_ref, w_ref, *o_refs):
    agg = jnp.concatenate([m0_ref[...], m1_ref[...],
                           m2_ref[...], m3_ref[...]], axis=1)
    sin = lax.rsqrt(jnp.maximum(ddst_ref[:, 0:1], 1.0))
    sout = lax.rsqrt(jnp.maximum(dsrc_ref[:, 0:1], 1.0))
    h = jnp.maximum(agg * sin + b_ref[...], 0.0)
    x = jnp.dot(h * sout, w_ref[...], preferred_element_type=jnp.float32)
    for j, o_ref in enumerate(o_refs):
        o_ref[...] = x[:, j * Q:(j + 1) * Q]


_tc2 = pl.pallas_call(
    _tc2_body,
    grid=(GRID,),
    in_specs=[_quarter_spec] * 4 + [
        _deg_spec,
        _deg_spec,
        pl.BlockSpec((1, D), lambda i: (0, 0)),
        pl.BlockSpec((D, D), lambda i: (0, 0)),
    ],
    out_specs=[_quarter_spec] * 4,
    out_shape=[jax.ShapeDtypeStruct((NP, Q), jnp.float32)] * 4,
)


def _tc3_body(m0_ref, m1_ref, m2_ref, m3_ref, ddst_ref, b_ref,
              oh_ref, og_ref):
    agg = jnp.concatenate([m0_ref[...], m1_ref[...],
                           m2_ref[...], m3_ref[...]], axis=1)
    sin = lax.rsqrt(jnp.maximum(ddst_ref[:, 0:1], 1.0))
    h = jnp.maximum(agg * sin + b_ref[...], 0.0)
    oh_ref[...] = h
    i = pl.program_id(0)
    rows = lax.broadcasted_iota(jnp.int32, (B, 1), 0) + i * B
    part = jnp.sum(jnp.where(rows < N, h, 0.0), axis=0, keepdims=True)

    @pl.when(i == 0)
    def _():
        og_ref[...] = jnp.zeros_like(og_ref)

    og_ref[...] += part

    @pl.when(i == GRID - 1)
    def _():
        og_ref[...] = og_ref[...] * (1.0 / N)


_tc3 = pl.pallas_call(
    _tc3_body,
    grid=(GRID,),
    in_specs=[_quarter_spec] * 4 + [
        _deg_spec,
        pl.BlockSpec((1, D), lambda i: (0, 0)),
    ],
    out_specs=[
        pl.BlockSpec((B, D), lambda i: (i, 0)),
        pl.BlockSpec((1, D), lambda i: (0, 0)),
    ],
    out_shape=[
        jax.ShapeDtypeStruct((NP, D), jnp.float32),
        jax.ShapeDtypeStruct((1, D), jnp.float32),
    ],
)


@jax.jit
def kernel(word_ids, ml, f, lf, ll, edge_index, emb_table, W1, b1, W2, b2):
    tbl = jnp.pad(emb_table, ((0, 0), (0, D - emb_table.shape[1])))
    wids3 = jnp.concatenate(
        [word_ids.astype(jnp.int32).reshape(-1),
         jnp.ones(((NP - N) * L,), jnp.int32)]).reshape(32, NW, L)
    ei = edge_index.astype(jnp.int32)
    pad_e = jnp.full((EP - E,), TRASH, jnp.int32)
    esrc3 = jnp.concatenate([ei[0], pad_e]).reshape(16, ECH, C)
    edst3 = jnp.concatenate([ei[1], pad_e]).reshape(16, ECH, C)
    scal = jnp.stack([ml, f, lf, ll], axis=1)
    scal = jnp.concatenate(
        [scal,
         jnp.concatenate([jnp.ones((NP - N, 1), jnp.float32),
                          jnp.zeros((NP - N, 3), jnp.float32)], axis=1)])
    z20 = jnp.zeros((L, D), jnp.float32)
    z16 = jnp.zeros((C, 16), jnp.float32)
    ones16 = jnp.ones((C, 16), jnp.float32)
    zq = jnp.zeros((C, Q), jnp.float32)

    we, dsrc, ddst = _embdeg(tbl, wids3, esrc3, edst3, z20, z16, ones16)
    x = _tc1(we, scal, dsrc, W1)
    m = _msg(*x, esrc3, edst3, zq)
    y = _tc2(*m, dsrc, ddst, b1.reshape(1, D), W2)
    n = _msg(*y, esrc3, edst3, zq)
    h, hg = _tc3(*n, ddst, b2.reshape(1, D))
    return h[:N], hg
